# baseline (speedup 1.0000x reference)
import jax
import jax.numpy as jnp
from jax.experimental import pallas as pl
from jax.experimental.pallas import tpu as pltpu

TAU = 2.0
V_TH = 1.0

S_TILE = 1024
T_CHUNK = 4
X_CHUNK = 8


def _ssm_kernel(x_ref, At_ref, Bt_ref, Ct_ref, D_ref, out_ref,
                h_ref, vs_ref, vo_ref):
    t2 = pl.program_id(2)

    @pl.when(t2 == 0)
    def _init():
        h_ref[...] = jnp.zeros_like(h_ref)
        vs_ref[...] = jnp.zeros_like(vs_ref)
        vo_ref[...] = jnp.zeros_like(vo_ref)

    At = At_ref[...]
    Bt = Bt_ref[...]
    Ct = Ct_ref[...]
    D = D_ref[...]

    h = h_ref[...]
    vs = vs_ref[...]
    vo = vo_ref[...]

    xoff = (t2 % (X_CHUNK // T_CHUNK)) * T_CHUNK

    for tt in range(T_CHUNK):
        xt = x_ref[0, xoff + tt]
        su = (jnp.dot(h, At, preferred_element_type=jnp.float32)
              + jnp.dot(xt, Bt, preferred_element_type=jnp.float32))
        vs = vs + (su - vs) * 0.5
        ms = vs >= V_TH
        s = ms.astype(jnp.float32)
        vs = jnp.where(ms, 0.0, vs)
        ou = jnp.dot(s, Ct, preferred_element_type=jnp.float32) + xt + D
        vo = vo + (ou - vo) * 0.5
        mo = vo >= V_TH
        so = mo.astype(jnp.float32)
        vo = jnp.where(mo, 0.0, vo)
        out_ref[0, tt] = so
        h = s

    h_ref[...] = h
    vs_ref[...] = vs
    vo_ref[...] = vo


def kernel(x, A, B, C, D):
    Bsz, T, S, d_model = x.shape
    d_state = A.shape[0]
    At = A.T
    Bt = B.T
    Ct = C.T
    D2 = D.reshape(1, d_model)

    ratio = X_CHUNK // T_CHUNK
    grid = (Bsz, S // S_TILE, T // T_CHUNK)
    return pl.pallas_call(
        _ssm_kernel,
        grid=grid,
        in_specs=[
            pl.BlockSpec((1, X_CHUNK, S_TILE, d_model),
                         lambda b, s, t: (b, t // ratio, s, 0)),
            pl.BlockSpec((d_state, d_state), lambda b, s, t: (0, 0)),
            pl.BlockSpec((d_model, d_state), lambda b, s, t: (0, 0)),
            pl.BlockSpec((d_state, d_model), lambda b, s, t: (0, 0)),
            pl.BlockSpec((1, d_model), lambda b, s, t: (0, 0)),
        ],
        out_specs=pl.BlockSpec((1, T_CHUNK, S_TILE, d_model),
                               lambda b, s, t: (b, t, s, 0)),
        out_shape=jax.ShapeDtypeStruct((Bsz, T, S, d_model), jnp.float32),
        scratch_shapes=[
            pltpu.VMEM((S_TILE, d_state), jnp.float32),
            pltpu.VMEM((S_TILE, d_state), jnp.float32),
            pltpu.VMEM((S_TILE, d_model), jnp.float32),
        ],
        compiler_params=pltpu.CompilerParams(
            dimension_semantics=("parallel", "parallel", "arbitrary"),
            vmem_limit_bytes=56 * 1024 * 1024,
        ),
    )(x, At, Bt, Ct, D2)


# speedup vs baseline: 1.2761x; 1.2761x over previous
import jax
import jax.numpy as jnp
from jax.experimental import pallas as pl
from jax.experimental.pallas import tpu as pltpu

TAU = 2.0
V_TH = 1.0

S_TILE = 1024
T_CHUNK = 2


def _ssm_kernel(x_ref, At_ref, Bt_ref, Ct_ref, D_ref, out_ref,
                h_ref, vs_ref, vo_ref):
    t2 = pl.program_id(2)

    @pl.when(t2 == 0)
    def _init():
        h_ref[...] = jnp.zeros_like(h_ref)
        vs_ref[...] = jnp.zeros_like(vs_ref)
        vo_ref[...] = jnp.zeros_like(vo_ref)

    At = At_ref[...]
    Bt = Bt_ref[...]
    Ct = Ct_ref[...]
    D = D_ref[...]

    h = h_ref[...]
    vs = vs_ref[...]
    vo = vo_ref[...]

    for tt in range(T_CHUNK):
        xt = x_ref[0, tt]
        su = (jnp.dot(h, At, preferred_element_type=jnp.float32)
              + jnp.dot(xt, Bt, preferred_element_type=jnp.float32))
        vs = vs + (su - vs) * 0.5
        ms = vs >= V_TH
        s = ms.astype(jnp.float32)
        vs = jnp.where(ms, 0.0, vs)
        ou = jnp.dot(s, Ct, preferred_element_type=jnp.float32) + xt + D
        vo = vo + (ou - vo) * 0.5
        mo = vo >= V_TH
        so = mo.astype(jnp.float32)
        vo = jnp.where(mo, 0.0, vo)
        out_ref[0, tt] = so
        h = s

    h_ref[...] = h
    vs_ref[...] = vs
    vo_ref[...] = vo


def kernel(x, A, B, C, D):
    Bsz, T, S, d_model = x.shape
    d_state = A.shape[0]
    At = A.T
    Bt = B.T
    Ct = C.T
    D2 = D.reshape(1, d_model)

    grid = (Bsz, S // S_TILE, T // T_CHUNK)
    return pl.pallas_call(
        _ssm_kernel,
        grid=grid,
        in_specs=[
            pl.BlockSpec((1, T_CHUNK, S_TILE, d_model),
                         lambda b, s, t: (b, t, s, 0)),
            pl.BlockSpec((d_state, d_state), lambda b, s, t: (0, 0)),
            pl.BlockSpec((d_model, d_state), lambda b, s, t: (0, 0)),
            pl.BlockSpec((d_state, d_model), lambda b, s, t: (0, 0)),
            pl.BlockSpec((1, d_model), lambda b, s, t: (0, 0)),
        ],
        out_specs=pl.BlockSpec((1, T_CHUNK, S_TILE, d_model),
                               lambda b, s, t: (b, t, s, 0)),
        out_shape=jax.ShapeDtypeStruct((Bsz, T, S, d_model), jnp.float32),
        scratch_shapes=[
            pltpu.VMEM((S_TILE, d_state), jnp.float32),
            pltpu.VMEM((S_TILE, d_state), jnp.float32),
            pltpu.VMEM((S_TILE, d_model), jnp.float32),
        ],
        compiler_params=pltpu.CompilerParams(
            dimension_semantics=("parallel", "parallel", "arbitrary"),
            vmem_limit_bytes=56 * 1024 * 1024,
        ),
    )(x, At, Bt, Ct, D2)


# speedup vs baseline: 1.4059x; 1.1017x over previous
import jax
import jax.numpy as jnp
from jax.experimental import pallas as pl
from jax.experimental.pallas import tpu as pltpu

TAU = 2.0
V_TH = 1.0

S_TILE = 1024
T_CHUNK = 4


def _ssm_kernel(x_ref, At_ref, Bt_ref, Ct_ref, D_ref, out_ref,
                h_ref, vs_ref, vo_ref):
    t2 = pl.program_id(2)

    @pl.when(t2 == 0)
    def _init():
        h_ref[...] = jnp.zeros_like(h_ref)
        vs_ref[...] = jnp.zeros_like(vs_ref)
        vo_ref[...] = jnp.zeros_like(vo_ref)

    At = At_ref[...]
    Bt = Bt_ref[...]
    Ct = Ct_ref[...]
    D = D_ref[...]

    h = h_ref[...]
    vs = vs_ref[...]
    vo = vo_ref[...]

    for tt in range(T_CHUNK):
        xt = x_ref[0, tt]
        su = (jnp.dot(h.astype(jnp.bfloat16), At, preferred_element_type=jnp.float32)
              + jnp.dot(xt.astype(jnp.bfloat16), Bt, preferred_element_type=jnp.float32))
        vs = vs + (su - vs) * 0.5
        ms = vs >= V_TH
        s = ms.astype(jnp.float32)
        vs = jnp.where(ms, 0.0, vs)
        ou = jnp.dot(s.astype(jnp.bfloat16), Ct, preferred_element_type=jnp.float32) + xt + D
        vo = vo + (ou - vo) * 0.5
        mo = vo >= V_TH
        so = mo.astype(jnp.float32)
        vo = jnp.where(mo, 0.0, vo)
        out_ref[0, tt] = so
        h = s

    h_ref[...] = h
    vs_ref[...] = vs
    vo_ref[...] = vo


def kernel(x, A, B, C, D):
    Bsz, T, S, d_model = x.shape
    d_state = A.shape[0]
    At = A.T
    Bt = B.T
    Ct = C.T
    D2 = D.reshape(1, d_model)

    grid = (Bsz, S // S_TILE, T // T_CHUNK)
    return pl.pallas_call(
        _ssm_kernel,
        grid=grid,
        in_specs=[
            pl.BlockSpec((1, T_CHUNK, S_TILE, d_model),
                         lambda b, s, t: (b, t, s, 0)),
            pl.BlockSpec((d_state, d_state), lambda b, s, t: (0, 0)),
            pl.BlockSpec((d_model, d_state), lambda b, s, t: (0, 0)),
            pl.BlockSpec((d_state, d_model), lambda b, s, t: (0, 0)),
            pl.BlockSpec((1, d_model), lambda b, s, t: (0, 0)),
        ],
        out_specs=pl.BlockSpec((1, T_CHUNK, S_TILE, d_model),
                               lambda b, s, t: (b, t, s, 0)),
        out_shape=jax.ShapeDtypeStruct((Bsz, T, S, d_model), jnp.float32),
        scratch_shapes=[
            pltpu.VMEM((S_TILE, d_state), jnp.float32),
            pltpu.VMEM((S_TILE, d_state), jnp.float32),
            pltpu.VMEM((S_TILE, d_model), jnp.float32),
        ],
        compiler_params=pltpu.CompilerParams(
            dimension_semantics=("parallel", "parallel", "arbitrary"),
            vmem_limit_bytes=56 * 1024 * 1024,
        ),
    )(x, At, Bt, Ct, D2)
